# baseline (speedup 1.0000x reference)
import jax
import jax.numpy as jnp
import numpy as np
from jax import lax
from jax.experimental import pallas as pl
from jax.experimental.pallas import tpu as pltpu
from jax.experimental.pallas import tpu_sc as plsc

B = 4
H = 512
W = 512
N = H * W
NC = 2
NS = 16
LANES = 16
PER_TILE = N // NS
CH = 2048
CROWS = CH // 128
CHUNKS = PER_TILE // CH
IMGS_PER_CORE = B // NC
LAG = 8
_CMAX = float(np.nextafter(np.float32(W), np.float32(0)))


def _sc_body(fxyd_hbm, zeros_hbm,
             out_a, out_b, out_c,
             fx_v, fy_v, dep_v, idx_buf, val_a, val_b, val_c,
             acc0_a, acc0_b, acc0_c, acc1_a, acc1_b, acc1_c,
             sem_scat, sem_load, sem_zero, sem_flush):
    c = lax.axis_index("c")
    s = lax.axis_index("s")
    tile_base = s * PER_TILE
    lane = lax.iota(jnp.int32, LANES)

    def load_chunk(b, q):
        csl = pl.ds(tile_base + q * CH, CH)
        qb = q % 2
        pltpu.async_copy(fxyd_hbm.at[0, b, csl], fx_v.at[qb], sem_load)
        pltpu.async_copy(fxyd_hbm.at[1, b, csl], fy_v.at[qb], sem_load)
        pltpu.async_copy(fxyd_hbm.at[2, b, csl], dep_v.at[qb], sem_load)

    def wait_chunk(b, q):
        csl = pl.ds(tile_base + q * CH, CH)
        qb = q % 2
        pltpu.make_async_copy(fxyd_hbm.at[0, b, csl], fx_v.at[qb], sem_load).wait()
        pltpu.make_async_copy(fxyd_hbm.at[1, b, csl], fy_v.at[qb], sem_load).wait()
        pltpu.make_async_copy(fxyd_hbm.at[2, b, csl], dep_v.at[qb], sem_load).wait()

    sl = pl.ds(tile_base, PER_TILE)
    acc_sets = ((acc0_a, acc0_b, acc0_c), (acc1_a, acc1_b, acc1_c))

    def zero_accs(ab):
        for acc in acc_sets[ab]:
            pltpu.async_copy(zeros_hbm, acc.at[sl], sem_zero)

    def wait_zero_accs(ab):
        for acc in acc_sets[ab]:
            pltpu.make_async_copy(zeros_hbm, acc.at[sl], sem_zero).wait()

    def flush_accs(ab, b, wait):
        for acc, out in zip(acc_sets[ab], (out_a, out_b, out_c)):
            cp = pltpu.make_async_copy(acc.at[sl], out.at[b, sl], sem_flush)
            if wait:
                cp.wait()
            else:
                cp.start()

    def scat_row(ab, r, enqueue):
        row = idx_buf.at[r]
        vsl = pl.ds(pl.multiple_of(r * 128, 128), 128)
        for acc, val in zip(acc_sets[ab], (val_a, val_b, val_c)):
            cp = pltpu.make_async_copy(val.at[vsl], acc.at[row], sem_scat)
            if enqueue:
                cp.start(add=True)
            else:
                cp.wait()

    def process_image(ab, b, next_b):
        for q in range(CHUNKS):
            wait_chunk(b, q)
            if q + 1 < CHUNKS:
                load_chunk(b, q + 1)
            elif next_b is not None:
                load_chunk(next_b, 0)
            px0 = tile_base + q * CH
            qb = q % 2

            @pl.loop(0, CROWS, unroll=2)
            def _row(r):
                for kk in range(8):
                    base = pl.multiple_of(r * 128 + kk * 16, 16)
                    fx16 = fx_v[qb, pl.ds(base, 16)]
                    fy16 = fy_v[qb, pl.ds(base, 16)]
                    d16 = dep_v[qb, pl.ds(base, 16)]
                    p0 = px0 + base
                    jf = (jnp.bitwise_and(p0, W - 1) + lane).astype(jnp.float32)
                    if_ = lax.shift_right_logical(p0, 9).astype(jnp.float32)
                    x2 = jf + fx16
                    y2 = if_ + fy16
                    valid = ((jnp.minimum(x2, y2) >= 0.0)
                             & (jnp.maximum(x2, y2) <= W - 1.0))
                    left = jnp.clip(x2, 0.0, _CMAX).astype(jnp.int32)
                    top = jnp.clip(y2, 0.0, _CMAX).astype(jnp.int32)
                    idx16 = lax.shift_left(top, 9) + left
                    w16 = jnp.where(valid, d16, 0.0)
                    idx_buf[r, pl.ds(kk * 16, 16)] = idx16
                    val_a[pl.ds(base, 16)] = fx16 * w16
                    val_b[pl.ds(base, 16)] = fy16 * w16
                    val_c[pl.ds(base, 16)] = w16
                scat_row(ab, r, True)

                @pl.when(r >= LAG)
                def _():
                    scat_row(ab, r - LAG, False)

            @pl.loop(CROWS - LAG, CROWS)
            def _drain(r):
                scat_row(ab, r, False)

    b0 = c * IMGS_PER_CORE
    b1 = b0 + 1
    zero_accs(0)
    load_chunk(b0, 0)
    wait_zero_accs(0)
    plsc.subcore_barrier()

    zero_accs(1)
    process_image(0, b0, b1)
    plsc.subcore_barrier()

    flush_accs(0, b0, wait=False)
    wait_zero_accs(1)
    plsc.subcore_barrier()

    process_image(1, b1, None)
    plsc.subcore_barrier()

    flush_accs(1, b1, wait=False)
    flush_accs(0, b0, wait=True)
    flush_accs(1, b1, wait=True)


@jax.jit
def _sc_scatter(fxyd, zeros):
    mesh = plsc.VectorSubcoreMesh(
        core_axis_name="c", subcore_axis_name="s",
        num_cores=NC, num_subcores=NS)
    f32 = jnp.float32
    return pl.kernel(
        _sc_body,
        out_type=(jax.ShapeDtypeStruct((B, N), f32),
                  jax.ShapeDtypeStruct((B, N), f32),
                  jax.ShapeDtypeStruct((B, N), f32)),
        mesh=mesh,
        scratch_types=[
            pltpu.VMEM((2, CH), f32),
            pltpu.VMEM((2, CH), f32),
            pltpu.VMEM((2, CH), f32),
            pltpu.VMEM((CROWS, 128), jnp.int32),
            pltpu.VMEM((CH,), f32),
            pltpu.VMEM((CH,), f32),
            pltpu.VMEM((CH,), f32),
            pltpu.VMEM_SHARED((N,), f32),
            pltpu.VMEM_SHARED((N,), f32),
            pltpu.VMEM_SHARED((N,), f32),
            pltpu.VMEM_SHARED((N,), f32),
            pltpu.VMEM_SHARED((N,), f32),
            pltpu.VMEM_SHARED((N,), f32),
            pltpu.SemaphoreType.DMA,
            pltpu.SemaphoreType.DMA,
            pltpu.SemaphoreType.DMA,
            pltpu.SemaphoreType.DMA,
        ],
    )(fxyd, zeros)


def _finish_body(afx_ref, afy_ref, acnt_ref, out_ref):
    col = lax.broadcasted_iota(jnp.int32, (H, W), 1)
    row = lax.broadcasted_iota(jnp.int32, (H, W), 0)

    def colpass(a):
        sh = pltpu.roll(a, 1, 1)
        sh = jnp.where(col == 0, 0.0, sh)
        return a + sh + jnp.where(col == W - 1, a, 0.0)

    def rowpass(cm):
        sh = pltpu.roll(cm, 1, 0)
        sh = jnp.where(row == 0, 0.0, sh)
        return cm + sh + jnp.where(row == H - 1, cm, 0.0)

    ofx = rowpass(colpass(afx_ref[0]))
    ofy = rowpass(colpass(afy_ref[0]))
    cnt = rowpass(colpass(acnt_ref[0]))
    safe = cnt > 0.0
    den = jnp.where(safe, cnt, 1.0)
    out_ref[0, 0] = -jnp.where(safe, ofx / den, ofx)
    out_ref[0, 1] = -jnp.where(safe, ofy / den, ofy)


@jax.jit
def _tc_finish(afx, afy, acnt):
    spec = pl.BlockSpec((1, H, W), lambda b: (b, 0, 0))
    return pl.pallas_call(
        _finish_body,
        grid=(B,),
        in_specs=[spec, spec, spec],
        out_specs=pl.BlockSpec((1, 2, H, W), lambda b: (b, 0, 0, 0)),
        out_shape=jax.ShapeDtypeStruct((B, 2, H, W), jnp.float32),
    )(afx, afy, acnt)


def kernel(input1, input2):
    fxyd = jnp.stack([input1[:, 0].reshape(B, N),
                      input1[:, 1].reshape(B, N),
                      input2[:, 0].reshape(B, N)])
    zeros = jnp.zeros((PER_TILE,), jnp.float32)
    a_fx, a_fy, a_cnt = _sc_scatter(fxyd, zeros)
    return _tc_finish(a_fx.reshape(B, H, W),
                      a_fy.reshape(B, H, W),
                      a_cnt.reshape(B, H, W))


# speedup vs baseline: 1.2343x; 1.2343x over previous
import jax
import jax.numpy as jnp
import numpy as np
from jax import lax
from jax.experimental import pallas as pl
from jax.experimental.pallas import tpu as pltpu
from jax.experimental.pallas import tpu_sc as plsc

B = 4
H = 512
W = 512
N = H * W
NC = 2
NS = 16
LANES = 16
PER_TILE = N // NS
CH = 2048
CROWS = CH // 128
CHUNKS = PER_TILE // CH
IMGS_PER_CORE = B // NC
LAG = 8
_CMAX = float(np.nextafter(np.float32(W), np.float32(0)))


def _sc_body(fx_hbm, fy_hbm, dep_hbm, zeros_hbm,
             out_a, out_b, out_c,
             fx_v, fy_v, dep_v, idx_buf, val_a, val_b, val_c,
             acc0_a, acc0_b, acc0_c, acc1_a, acc1_b, acc1_c,
             sem_scat, sem_load, sem_zero, sem_flush):
    c = lax.axis_index("c")
    s = lax.axis_index("s")
    tile_base = s * PER_TILE
    lane = lax.iota(jnp.int32, LANES)

    def load_chunk(b, q):
        csl = pl.ds(tile_base + q * CH, CH)
        qb = q % 2
        pltpu.async_copy(fx_hbm.at[b, csl], fx_v.at[qb], sem_load)
        pltpu.async_copy(fy_hbm.at[b, csl], fy_v.at[qb], sem_load)
        pltpu.async_copy(dep_hbm.at[b, csl], dep_v.at[qb], sem_load)

    def wait_chunk(b, q):
        csl = pl.ds(tile_base + q * CH, CH)
        qb = q % 2
        pltpu.make_async_copy(fx_hbm.at[b, csl], fx_v.at[qb], sem_load).wait()
        pltpu.make_async_copy(fy_hbm.at[b, csl], fy_v.at[qb], sem_load).wait()
        pltpu.make_async_copy(dep_hbm.at[b, csl], dep_v.at[qb], sem_load).wait()

    sl = pl.ds(tile_base, PER_TILE)
    acc_sets = ((acc0_a, acc0_b, acc0_c), (acc1_a, acc1_b, acc1_c))

    def zero_accs(ab):
        for acc in acc_sets[ab]:
            pltpu.async_copy(zeros_hbm, acc.at[sl], sem_zero)

    def wait_zero_accs(ab):
        for acc in acc_sets[ab]:
            pltpu.make_async_copy(zeros_hbm, acc.at[sl], sem_zero).wait()

    def flush_accs(ab, b, wait):
        for acc, out in zip(acc_sets[ab], (out_a, out_b, out_c)):
            cp = pltpu.make_async_copy(acc.at[sl], out.at[b, sl], sem_flush)
            if wait:
                cp.wait()
            else:
                cp.start()

    def scat_row(ab, r, enqueue):
        row = idx_buf.at[r]
        vsl = pl.ds(pl.multiple_of(r * 128, 128), 128)
        for acc, val in zip(acc_sets[ab], (val_a, val_b, val_c)):
            cp = pltpu.make_async_copy(val.at[vsl], acc.at[row], sem_scat)
            if enqueue:
                cp.start(add=True)
            else:
                cp.wait()

    def process_image(ab, b, next_b):
        for q in range(CHUNKS):
            wait_chunk(b, q)
            if q + 1 < CHUNKS:
                load_chunk(b, q + 1)
            elif next_b is not None:
                load_chunk(next_b, 0)
            px0 = tile_base + q * CH
            qb = q % 2

            @pl.loop(0, CROWS)
            def _row(r):
                for kk in range(8):
                    base = pl.multiple_of(r * 128 + kk * 16, 16)
                    fx16 = fx_v[qb, pl.ds(base, 16)]
                    fy16 = fy_v[qb, pl.ds(base, 16)]
                    d16 = dep_v[qb, pl.ds(base, 16)]
                    p0 = px0 + base
                    jf = (jnp.bitwise_and(p0, W - 1) + lane).astype(jnp.float32)
                    if_ = lax.shift_right_logical(p0, 9).astype(jnp.float32)
                    x2 = jf + fx16
                    y2 = if_ + fy16
                    valid = ((jnp.minimum(x2, y2) >= 0.0)
                             & (jnp.maximum(x2, y2) <= W - 1.0))
                    left = jnp.clip(x2, 0.0, _CMAX).astype(jnp.int32)
                    top = jnp.clip(y2, 0.0, _CMAX).astype(jnp.int32)
                    idx16 = lax.shift_left(top, 9) + left
                    w16 = jnp.where(valid, d16, 0.0)
                    idx_buf[r, pl.ds(kk * 16, 16)] = idx16
                    val_a[pl.ds(base, 16)] = fx16 * w16
                    val_b[pl.ds(base, 16)] = fy16 * w16
                    val_c[pl.ds(base, 16)] = w16
                scat_row(ab, r, True)

                @pl.when(r >= LAG)
                def _():
                    scat_row(ab, r - LAG, False)

            @pl.loop(CROWS - LAG, CROWS)
            def _drain(r):
                scat_row(ab, r, False)

    b0 = c * IMGS_PER_CORE
    b1 = b0 + 1
    zero_accs(0)
    load_chunk(b0, 0)
    wait_zero_accs(0)
    plsc.subcore_barrier()

    zero_accs(1)
    process_image(0, b0, b1)
    plsc.subcore_barrier()

    flush_accs(0, b0, wait=False)
    wait_zero_accs(1)
    plsc.subcore_barrier()

    process_image(1, b1, None)
    plsc.subcore_barrier()

    flush_accs(1, b1, wait=False)
    flush_accs(0, b0, wait=True)
    flush_accs(1, b1, wait=True)


@jax.jit
def _sc_scatter(fx, fy, dep, zeros):
    mesh = plsc.VectorSubcoreMesh(
        core_axis_name="c", subcore_axis_name="s",
        num_cores=NC, num_subcores=NS)
    f32 = jnp.float32
    return pl.kernel(
        _sc_body,
        out_type=(jax.ShapeDtypeStruct((B, N), f32),
                  jax.ShapeDtypeStruct((B, N), f32),
                  jax.ShapeDtypeStruct((B, N), f32)),
        mesh=mesh,
        scratch_types=[
            pltpu.VMEM((2, CH), f32),
            pltpu.VMEM((2, CH), f32),
            pltpu.VMEM((2, CH), f32),
            pltpu.VMEM((CROWS, 128), jnp.int32),
            pltpu.VMEM((CH,), f32),
            pltpu.VMEM((CH,), f32),
            pltpu.VMEM((CH,), f32),
            pltpu.VMEM_SHARED((N,), f32),
            pltpu.VMEM_SHARED((N,), f32),
            pltpu.VMEM_SHARED((N,), f32),
            pltpu.VMEM_SHARED((N,), f32),
            pltpu.VMEM_SHARED((N,), f32),
            pltpu.VMEM_SHARED((N,), f32),
            pltpu.SemaphoreType.DMA,
            pltpu.SemaphoreType.DMA,
            pltpu.SemaphoreType.DMA,
            pltpu.SemaphoreType.DMA,
        ],
    )(fx, fy, dep, zeros)


def _finish_body(afx_ref, afy_ref, acnt_ref, out_ref):
    col = lax.broadcasted_iota(jnp.int32, (H, W), 1)
    row = lax.broadcasted_iota(jnp.int32, (H, W), 0)

    def colpass(a):
        sh = pltpu.roll(a, 1, 1)
        sh = jnp.where(col == 0, 0.0, sh)
        return a + sh + jnp.where(col == W - 1, a, 0.0)

    def rowpass(cm):
        sh = pltpu.roll(cm, 1, 0)
        sh = jnp.where(row == 0, 0.0, sh)
        return cm + sh + jnp.where(row == H - 1, cm, 0.0)

    ofx = rowpass(colpass(afx_ref[0]))
    ofy = rowpass(colpass(afy_ref[0]))
    cnt = rowpass(colpass(acnt_ref[0]))
    safe = cnt > 0.0
    den = jnp.where(safe, cnt, 1.0)
    out_ref[0, 0] = -jnp.where(safe, ofx / den, ofx)
    out_ref[0, 1] = -jnp.where(safe, ofy / den, ofy)


@jax.jit
def _tc_finish(afx, afy, acnt):
    spec = pl.BlockSpec((1, H, W), lambda b: (b, 0, 0))
    return pl.pallas_call(
        _finish_body,
        grid=(B,),
        in_specs=[spec, spec, spec],
        out_specs=pl.BlockSpec((1, 2, H, W), lambda b: (b, 0, 0, 0)),
        out_shape=jax.ShapeDtypeStruct((B, 2, H, W), jnp.float32),
    )(afx, afy, acnt)


def kernel(input1, input2):
    fx = input1[:, 0].reshape(B, N)
    fy = input1[:, 1].reshape(B, N)
    dep = input2[:, 0].reshape(B, N)
    zeros = jnp.zeros((PER_TILE,), jnp.float32)
    a_fx, a_fy, a_cnt = _sc_scatter(fx, fy, dep, zeros)
    return _tc_finish(a_fx.reshape(B, H, W),
                      a_fy.reshape(B, H, W),
                      a_cnt.reshape(B, H, W))


# speedup vs baseline: 1.2561x; 1.0176x over previous
import jax
import jax.numpy as jnp
import numpy as np
from jax import lax
from jax.experimental import pallas as pl
from jax.experimental.pallas import tpu as pltpu
from jax.experimental.pallas import tpu_sc as plsc

B = 4
H = 512
W = 512
N = H * W
NC = 2
NS = 16
LANES = 16
PER_TILE = N // NS
CH = 2048
CROWS = CH // 128
CHUNKS = PER_TILE // CH
IMGS_PER_CORE = B // NC
LAG = 8
_CMAX = float(np.nextafter(np.float32(W), np.float32(0)))


def _sc_body(fx_hbm, fy_hbm, dep_hbm, zeros_hbm,
             out_a, out_b, out_c,
             fx_v, fy_v, dep_v, idx_buf, val_a, val_b, val_c,
             acc0_a, acc0_b, acc0_c, acc1_a, acc1_b, acc1_c,
             sem_scat, sem_load, sem_zero, sem_flush):
    c = lax.axis_index("c")
    s = lax.axis_index("s")
    tile_base = s * PER_TILE
    lane = lax.iota(jnp.int32, LANES)

    def load_chunk(b, q):
        csl = pl.ds(tile_base + q * CH, CH)
        qb = q % 2
        pltpu.async_copy(fx_hbm.at[b, csl], fx_v.at[qb], sem_load)
        pltpu.async_copy(fy_hbm.at[b, csl], fy_v.at[qb], sem_load)
        pltpu.async_copy(dep_hbm.at[b, csl], dep_v.at[qb], sem_load)

    def wait_chunk(b, q):
        csl = pl.ds(tile_base + q * CH, CH)
        qb = q % 2
        pltpu.make_async_copy(fx_hbm.at[b, csl], fx_v.at[qb], sem_load).wait()
        pltpu.make_async_copy(fy_hbm.at[b, csl], fy_v.at[qb], sem_load).wait()
        pltpu.make_async_copy(dep_hbm.at[b, csl], dep_v.at[qb], sem_load).wait()

    sl = pl.ds(tile_base, PER_TILE)
    acc_sets = ((acc0_a, acc0_b, acc0_c), (acc1_a, acc1_b, acc1_c))

    def zero_accs(ab):
        for acc in acc_sets[ab]:
            pltpu.async_copy(zeros_hbm, acc.at[sl], sem_zero)

    def wait_zero_accs(ab):
        for acc in acc_sets[ab]:
            pltpu.make_async_copy(zeros_hbm, acc.at[sl], sem_zero).wait()

    ROWS_PER_TILE = PER_TILE // W
    row0 = s * ROWS_PER_TILE

    def flush_accs(ab, b, wait):
        for acc, out in zip(acc_sets[ab], (out_a, out_b, out_c)):
            for r in range(ROWS_PER_TILE):
                cp = pltpu.make_async_copy(
                    acc.at[pl.ds(tile_base + r * W, W)],
                    out.at[b, row0 + r], sem_flush)
                if wait:
                    cp.wait()
                else:
                    cp.start()

    def scat_row(ab, r, enqueue):
        row = idx_buf.at[r]
        vsl = pl.ds(pl.multiple_of(r * 128, 128), 128)
        for acc, val in zip(acc_sets[ab], (val_a, val_b, val_c)):
            cp = pltpu.make_async_copy(val.at[vsl], acc.at[row], sem_scat)
            if enqueue:
                cp.start(add=True)
            else:
                cp.wait()

    def process_image(ab, b, next_b):
        for q in range(CHUNKS):
            wait_chunk(b, q)
            if q + 1 < CHUNKS:
                load_chunk(b, q + 1)
            elif next_b is not None:
                load_chunk(next_b, 0)
            px0 = tile_base + q * CH
            qb = q % 2

            @pl.loop(0, CROWS)
            def _row(r):
                for kk in range(8):
                    base = pl.multiple_of(r * 128 + kk * 16, 16)
                    fx16 = fx_v[qb, pl.ds(base, 16)]
                    fy16 = fy_v[qb, pl.ds(base, 16)]
                    d16 = dep_v[qb, pl.ds(base, 16)]
                    p0 = px0 + base
                    jf = (jnp.bitwise_and(p0, W - 1) + lane).astype(jnp.float32)
                    if_ = lax.shift_right_logical(p0, 9).astype(jnp.float32)
                    x2 = jf + fx16
                    y2 = if_ + fy16
                    valid = ((jnp.minimum(x2, y2) >= 0.0)
                             & (jnp.maximum(x2, y2) <= W - 1.0))
                    left = jnp.clip(x2, 0.0, _CMAX).astype(jnp.int32)
                    top = jnp.clip(y2, 0.0, _CMAX).astype(jnp.int32)
                    idx16 = lax.shift_left(top, 9) + left
                    w16 = jnp.where(valid, d16, 0.0)
                    idx_buf[r, pl.ds(kk * 16, 16)] = idx16
                    val_a[pl.ds(base, 16)] = fx16 * w16
                    val_b[pl.ds(base, 16)] = fy16 * w16
                    val_c[pl.ds(base, 16)] = w16
                scat_row(ab, r, True)

                @pl.when(r >= LAG)
                def _():
                    scat_row(ab, r - LAG, False)

            @pl.loop(CROWS - LAG, CROWS)
            def _drain(r):
                scat_row(ab, r, False)

    b0 = c * IMGS_PER_CORE
    b1 = b0 + 1
    zero_accs(0)
    load_chunk(b0, 0)
    wait_zero_accs(0)
    plsc.subcore_barrier()

    zero_accs(1)
    process_image(0, b0, b1)
    plsc.subcore_barrier()

    flush_accs(0, b0, wait=False)
    wait_zero_accs(1)
    plsc.subcore_barrier()

    process_image(1, b1, None)
    plsc.subcore_barrier()

    flush_accs(1, b1, wait=False)
    flush_accs(0, b0, wait=True)
    flush_accs(1, b1, wait=True)


@jax.jit
def _sc_scatter(fx, fy, dep, zeros):
    mesh = plsc.VectorSubcoreMesh(
        core_axis_name="c", subcore_axis_name="s",
        num_cores=NC, num_subcores=NS)
    f32 = jnp.float32
    return pl.kernel(
        _sc_body,
        out_type=(jax.ShapeDtypeStruct((B, H, W), f32),
                  jax.ShapeDtypeStruct((B, H, W), f32),
                  jax.ShapeDtypeStruct((B, H, W), f32)),
        mesh=mesh,
        scratch_types=[
            pltpu.VMEM((2, CH), f32),
            pltpu.VMEM((2, CH), f32),
            pltpu.VMEM((2, CH), f32),
            pltpu.VMEM((CROWS, 128), jnp.int32),
            pltpu.VMEM((CH,), f32),
            pltpu.VMEM((CH,), f32),
            pltpu.VMEM((CH,), f32),
            pltpu.VMEM_SHARED((N,), f32),
            pltpu.VMEM_SHARED((N,), f32),
            pltpu.VMEM_SHARED((N,), f32),
            pltpu.VMEM_SHARED((N,), f32),
            pltpu.VMEM_SHARED((N,), f32),
            pltpu.VMEM_SHARED((N,), f32),
            pltpu.SemaphoreType.DMA,
            pltpu.SemaphoreType.DMA,
            pltpu.SemaphoreType.DMA,
            pltpu.SemaphoreType.DMA,
        ],
    )(fx, fy, dep, zeros)


def _finish_body(afx_ref, afy_ref, acnt_ref, out_ref):
    col = lax.broadcasted_iota(jnp.int32, (H, W), 1)
    row = lax.broadcasted_iota(jnp.int32, (H, W), 0)

    def colpass(a):
        sh = pltpu.roll(a, 1, 1)
        sh = jnp.where(col == 0, 0.0, sh)
        return a + sh + jnp.where(col == W - 1, a, 0.0)

    def rowpass(cm):
        sh = pltpu.roll(cm, 1, 0)
        sh = jnp.where(row == 0, 0.0, sh)
        return cm + sh + jnp.where(row == H - 1, cm, 0.0)

    ofx = rowpass(colpass(afx_ref[0]))
    ofy = rowpass(colpass(afy_ref[0]))
    cnt = rowpass(colpass(acnt_ref[0]))
    safe = cnt > 0.0
    den = jnp.where(safe, cnt, 1.0)
    out_ref[0, 0] = -jnp.where(safe, ofx / den, ofx)
    out_ref[0, 1] = -jnp.where(safe, ofy / den, ofy)


@jax.jit
def _tc_finish(afx, afy, acnt):
    spec = pl.BlockSpec((1, H, W), lambda b: (b, 0, 0))
    return pl.pallas_call(
        _finish_body,
        grid=(B,),
        in_specs=[spec, spec, spec],
        out_specs=pl.BlockSpec((1, 2, H, W), lambda b: (b, 0, 0, 0)),
        out_shape=jax.ShapeDtypeStruct((B, 2, H, W), jnp.float32),
    )(afx, afy, acnt)


def kernel(input1, input2):
    fx = input1[:, 0].reshape(B, N)
    fy = input1[:, 1].reshape(B, N)
    dep = input2[:, 0].reshape(B, N)
    zeros = jnp.zeros((PER_TILE,), jnp.float32)
    a_fx, a_fy, a_cnt = _sc_scatter(fx, fy, dep, zeros)
    return _tc_finish(a_fx, a_fy, a_cnt)
